# Initial kernel scaffold; baseline (speedup 1.0000x reference)
import jax
import jax.numpy as jnp
from jax.experimental import pallas as pl


def kernel(batch_x, batch_x_time_stamp, batch_y, batch_y_time_stamp, batch_c, params):
    raise NotImplementedError("write your pallas kernel here")


# speedup vs baseline: 3.5784x; 3.5784x over previous
import functools
import math

import numpy as np
import jax
import jax.numpy as jnp
from jax.experimental import pallas as pl
from jax.experimental.pallas import tpu as pltpu

L = 2048
D_MODEL = 768
N_HEADS = 12
D_HEAD = 64
D_FF = 3072
N_LAYERS = 2
FACTOR = 5
U_TOP = min(int(FACTOR * math.ceil(math.log(L))), L)


def _pos_embedding_np(seq_len, d_model):
    pe = np.zeros((seq_len, d_model), dtype=np.float32)
    position = np.arange(seq_len, dtype=np.float32)[:, None]
    div_term = np.exp(np.arange(0, d_model, 2, dtype=np.float32)
                      * -(math.log(10000.0) / d_model))
    pe[:, 0::2] = np.sin(position * div_term)
    pe[:, 1::2] = np.cos(position * div_term)
    return pe


def _sample_count_matrices():
    mats = []
    cpu = jax.devices("cpu")[0]
    with jax.default_device(cpu):
        for layer in range(N_LAYERS):
            skey = jax.random.key(1000 + layer)
            idx = np.asarray(jax.random.randint(skey, (L, U_TOP), 0, L))
            pt = np.zeros((L, L), dtype=np.int8)
            np.add.at(pt, (idx, np.arange(L)[:, None]), 1)
            mats.append(pt)
    return mats


_PE = _pos_embedding_np(L, D_MODEL)
_PTS = _sample_count_matrices()


def _mm_body(do_gelu, do_ln, has_res, pre_ln, *refs):
    if pre_ln:
        x_ref, w_ref, b_ref, g_ref, bl_ref, o_ref = refs
    elif do_ln and has_res:
        x_ref, w_ref, b_ref, r_ref, g_ref, bl_ref, o_ref = refs
    elif has_res:
        x_ref, w_ref, b_ref, r_ref, o_ref = refs
    else:
        x_ref, w_ref, b_ref, o_ref = refs

    x = x_ref[...]
    if pre_ln:
        m = jnp.mean(x, axis=-1, keepdims=True)
        d = x - m
        v = jnp.mean(d * d, axis=-1, keepdims=True)
        x = d / jnp.sqrt(v + 1e-5) * g_ref[...] + bl_ref[...]
    acc = jax.lax.dot_general(x, w_ref[...], (((1,), (1,)), ((), ())),
                              preferred_element_type=jnp.float32)
    acc = acc + b_ref[...]
    if has_res:
        acc = acc + r_ref[...]
    if do_gelu:
        acc = 0.5 * acc * (1.0 + jax.lax.erf(acc / np.sqrt(2.0).astype(np.float32)))
    if do_ln and not pre_ln:
        m = jnp.mean(acc, axis=-1, keepdims=True)
        d = acc - m
        v = jnp.mean(d * d, axis=-1, keepdims=True)
        acc = d / jnp.sqrt(v + 1e-5) * g_ref[...] + bl_ref[...]
    o_ref[...] = acc


def _mm(x, w, b, res=None, ln=None, do_gelu=False, pre_ln=False, bm=512):
    mrows, k = x.shape
    n = w.shape[0]
    grid = (mrows // bm,)
    in_specs = [
        pl.BlockSpec((bm, k), lambda i: (i, 0)),
        pl.BlockSpec((n, k), lambda i: (0, 0)),
        pl.BlockSpec((1, n), lambda i: (0, 0)),
    ]
    args = [x, w, b.reshape(1, n)]
    if res is not None:
        in_specs.append(pl.BlockSpec((bm, n), lambda i: (i, 0)))
        args.append(res)
    if ln is not None:
        g, bl = ln
        dln = k if pre_ln else n
        in_specs += [pl.BlockSpec((1, dln), lambda i: (0, 0))] * 2
        args += [g.reshape(1, dln), bl.reshape(1, dln)]
    body = functools.partial(_mm_body, do_gelu, ln is not None and not pre_ln,
                             res is not None, pre_ln)
    return pl.pallas_call(
        body,
        grid=grid,
        in_specs=in_specs,
        out_specs=pl.BlockSpec((bm, n), lambda i: (i, 0)),
        out_shape=jax.ShapeDtypeStruct((mrows, n), jnp.float32),
        compiler_params=pltpu.CompilerParams(
            dimension_semantics=("parallel",)),
    )(*args)


def _qkv_body(x_ref, wq_ref, wk_ref, wv_ref, bq_ref, bk_ref, bv_ref,
              q_ref, k_ref, v_ref):
    x = x_ref[...]
    dn = (((1,), (1,)), ((), ()))
    q_ref[...] = jax.lax.dot_general(x, wq_ref[...], dn,
                                     preferred_element_type=jnp.float32) + bq_ref[...]
    k_ref[...] = jax.lax.dot_general(x, wk_ref[...], dn,
                                     preferred_element_type=jnp.float32) + bk_ref[...]
    v_ref[...] = jax.lax.dot_general(x, wv_ref[...], dn,
                                     preferred_element_type=jnp.float32) + bv_ref[...]


def _qkv(x, wq, wk, wv, bq, bk, bv, bm=512):
    grid = (L // bm,)
    wspec = pl.BlockSpec((D_MODEL, D_MODEL), lambda i: (0, 0))
    bspec = pl.BlockSpec((1, D_MODEL), lambda i: (0, 0))
    ospec = pl.BlockSpec((bm, D_MODEL), lambda i: (i, 0))
    oshape = jax.ShapeDtypeStruct((L, D_MODEL), jnp.float32)
    return pl.pallas_call(
        _qkv_body,
        grid=grid,
        in_specs=[pl.BlockSpec((bm, D_MODEL), lambda i: (i, 0)),
                  wspec, wspec, wspec, bspec, bspec, bspec],
        out_specs=(ospec, ospec, ospec),
        out_shape=(oshape, oshape, oshape),
        compiler_params=pltpu.CompilerParams(
            dimension_semantics=("parallel",)),
    )(x, wq, wk, wv, bq.reshape(1, -1), bk.reshape(1, -1), bv.reshape(1, -1))


def _attn_body(q_ref, k_ref, v_ref, pt_ref, o_ref, onehot_ref):
    q = q_ref[0]
    k = k_ref[0]
    v = v_ref[0]
    dn_t = (((1,), (1,)), ((), ()))
    st = jax.lax.dot_general(k, q, dn_t, preferred_element_type=jnp.float32)
    pt = pt_ref[...].astype(jnp.float32)
    neg = jnp.where(pt > 0, 0.0, -1e30)
    m_max = jnp.max(st + neg, axis=0, keepdims=True)
    m_sum = jnp.sum(st * pt, axis=0, keepdims=True)
    m = m_max - m_sum * (1.0 / L)

    lane = jax.lax.broadcasted_iota(jnp.int32, (1, L), 1)

    def pick(i, m_cur):
        top = jnp.max(m_cur, axis=1, keepdims=True)
        pos = jnp.min(jnp.where(m_cur == top, lane, L + 1), axis=1,
                      keepdims=True)
        sel = (lane == pos)
        onehot_ref[pl.ds(i, 1), :] = sel.astype(jnp.float32)
        return jnp.where(sel, -1e30, m_cur)

    jax.lax.fori_loop(0, U_TOP, pick, m)
    o = onehot_ref[...]

    q_red = jax.lax.dot_general(o, q, (((1,), (0,)), ((), ())),
                                preferred_element_type=jnp.float32)
    sc = jax.lax.dot_general(q_red, k, dn_t,
                             preferred_element_type=jnp.float32)
    sc = sc * (1.0 / math.sqrt(D_HEAD))
    sc = sc - jnp.max(sc, axis=1, keepdims=True)
    e = jnp.exp(sc)
    attn = e / jnp.sum(e, axis=1, keepdims=True)
    upd = jax.lax.dot_general(attn, v, (((1,), (0,)), ((), ())),
                              preferred_element_type=jnp.float32)
    mean_v = jnp.mean(v, axis=0, keepdims=True)
    ctx = jax.lax.dot_general(o, upd, (((0,), (0,)), ((), ())),
                              preferred_element_type=jnp.float32)
    rowsel = jax.lax.dot_general(o, jnp.ones((U_TOP, 1), jnp.float32),
                                 (((0,), (0,)), ((), ())),
                                 preferred_element_type=jnp.float32)
    o_ref[0] = ctx + (1.0 - rowsel) * mean_v


def _prob_attn(q, k, v, pt):
    hspec = pl.BlockSpec((1, L, D_HEAD), lambda h: (h, 0, 0))
    return pl.pallas_call(
        _attn_body,
        grid=(N_HEADS,),
        in_specs=[hspec, hspec, hspec,
                  pl.BlockSpec((L, L), lambda h: (0, 0))],
        out_specs=hspec,
        out_shape=jax.ShapeDtypeStruct((N_HEADS, L, D_HEAD), jnp.float32),
        scratch_shapes=[pltpu.VMEM((U_TOP, L), jnp.float32)],
        compiler_params=pltpu.CompilerParams(
            dimension_semantics=("arbitrary",)),
    )(q, k, v, pt)


def kernel(batch_x, batch_x_time_stamp, batch_y, batch_y_time_stamp,
           batch_c, params):
    x_in = batch_x[0]
    mark = batch_x_time_stamp[0]
    enc_in = x_in.shape[1]
    mark_dim = mark.shape[1]

    x_cat = jnp.concatenate(
        [jnp.roll(x_in, 1, axis=0), x_in, jnp.roll(x_in, -1, axis=0), mark],
        axis=-1)
    kdim = 3 * enc_in + mark_dim
    kpad = 32
    x_cat = jnp.pad(x_cat, ((0, 0), (0, kpad - kdim)))
    wv = params['Wv_emb']
    w_cat = jnp.concatenate([wv[:, :, 0], wv[:, :, 1], wv[:, :, 2],
                             params['Wt_emb']], axis=1)
    w_cat = jnp.pad(w_cat, ((0, 0), (0, kpad - kdim)))
    pe = jnp.asarray(_PE)
    x = _mm(x_cat, w_cat, jnp.zeros((D_MODEL,), jnp.float32), res=pe)

    pts = _PTS
    for layer in range(N_LAYERS):
        p = params['layers'][layer]
        q, k, v = _qkv(x, p['Wq'], p['Wk'], p['Wv'], p['bq'], p['bk'], p['bv'])
        q3 = q.reshape(L, N_HEADS, D_HEAD).transpose(1, 0, 2)
        k3 = k.reshape(L, N_HEADS, D_HEAD).transpose(1, 0, 2)
        v3 = v.reshape(L, N_HEADS, D_HEAD).transpose(1, 0, 2)
        ctx = _prob_attn(q3, k3, v3, jnp.asarray(pts[layer]))
        ctx = ctx.transpose(1, 0, 2).reshape(L, D_MODEL)
        x = _mm(ctx, p['Wo'], p['bo'], res=x, ln=(p['g1'], p['b1']))
        y = _mm(x, p['Wc1'], p['bc1'], do_gelu=True)
        x = _mm(y, p['Wc2'], p['bc2'], res=x, ln=(p['g2'], p['b2']))

    c_out = params['Wproj'].shape[0]
    wp = jnp.pad(params['Wproj'], ((0, 128 - c_out), (0, 0)))
    bp = jnp.pad(params['bproj'], (0, 128 - c_out))
    out = _mm(x, wp, bp, ln=(params['norm_g'], params['norm_b']), pre_ln=True)
    return out[:, :c_out][None]
